# Initial kernel scaffold; baseline (speedup 1.0000x reference)
import jax
import jax.numpy as jnp
from jax.experimental import pallas as pl


def kernel(x, pe):
    raise NotImplementedError("write your pallas kernel here")


# speedup vs baseline: 3.2039x; 3.2039x over previous
import functools

import jax
import jax.numpy as jnp
from jax import lax
from jax.experimental import pallas as pl
from jax.experimental.pallas import tpu as pltpu
from jax.experimental.pallas import tpu_sc as plsc

B, L, D = 16384, 200, 128
BT = B * L
NC, NS = 2, 16
NW = NC * NS
BPW = BT // NW
GSUB = 128
NQ = 2
CH = NQ * GSUB
NCH = BPW // CH


def _body(x_hbm, tab_hbm, out_hbm,
          idx0, idx1, rows0, rows1,
          isem0, isem1, gsem0, gsem1, osem0, osem1):
    wid = lax.axis_index("s") * NC + lax.axis_index("c")
    base = wid * BPW
    idx = (idx0, idx1)
    rows = (rows0, rows1)
    isem = (isem0, isem1)
    gsem = (gsem0, gsem1)
    osem = (osem0, osem1)

    def fetch_idx(c, s):
        pltpu.async_copy(x_hbm.at[wid, c], idx[s], isem[s])

    def wait_idx(c, s):
        pltpu.make_async_copy(x_hbm.at[wid, c], idx[s], isem[s]).wait()

    def start_gathers(s):
        for q in range(NQ):
            pltpu.async_copy(tab_hbm.at[idx[s].at[q]],
                             rows[s].at[pl.ds(q * GSUB, GSUB)], gsem[s])

    def wait_gathers(s):
        for q in range(NQ):
            pltpu.make_async_copy(tab_hbm.at[idx[s].at[q]],
                                  rows[s].at[pl.ds(q * GSUB, GSUB)],
                                  gsem[s]).wait()

    def start_scatter(c, s):
        pltpu.async_copy(rows[s], out_hbm.at[pl.ds(base + c * CH, CH)],
                         osem[s])

    def wait_scatter(c, s):
        pltpu.make_async_copy(rows[s], out_hbm.at[pl.ds(base + c * CH, CH)],
                              osem[s]).wait()

    fetch_idx(0, 0)
    fetch_idx(1, 1)
    wait_idx(0, 0)
    start_gathers(0)

    def step(g, _):
        c0 = 2 * g
        wait_gathers(0)
        start_scatter(c0, 0)

        @pl.when(g <= NCH // 2 - 2)
        def _():
            fetch_idx(c0 + 2, 0)

        @pl.when(g >= 1)
        def _():
            wait_scatter(c0 - 1, 1)

        wait_idx(c0 + 1, 1)
        start_gathers(1)

        wait_gathers(1)
        start_scatter(c0 + 1, 1)

        @pl.when(g <= NCH // 2 - 2)
        def _():
            fetch_idx(c0 + 3, 1)
            wait_scatter(c0, 0)
            wait_idx(c0 + 2, 0)
            start_gathers(0)

        return 0

    lax.fori_loop(0, NCH // 2, step, 0)

    wait_scatter(NCH - 2, 0)
    wait_scatter(NCH - 1, 1)


_mesh = plsc.VectorSubcoreMesh(core_axis_name="c", subcore_axis_name="s")

_sc_gather = pl.kernel(
    _body,
    out_type=jax.ShapeDtypeStruct((BT, D), jnp.float32),
    mesh=_mesh,
    scratch_types=[
        pltpu.VMEM((NQ, GSUB), jnp.int32),
        pltpu.VMEM((NQ, GSUB), jnp.int32),
        pltpu.VMEM((CH, D), jnp.float32),
        pltpu.VMEM((CH, D), jnp.float32),
        pltpu.SemaphoreType.DMA,
        pltpu.SemaphoreType.DMA,
        pltpu.SemaphoreType.DMA,
        pltpu.SemaphoreType.DMA,
        pltpu.SemaphoreType.DMA,
        pltpu.SemaphoreType.DMA,
    ],
)


@jax.jit
def kernel(x, pe):
    xr = x.reshape(NW, NCH, NQ, GSUB)
    out = _sc_gather(xr, pe)
    return out.reshape(B, L, D)


# speedup vs baseline: 18.9548x; 5.9163x over previous
import functools

import jax
import jax.numpy as jnp
from jax import lax
from jax.experimental import pallas as pl
from jax.experimental.pallas import tpu as pltpu
from jax.experimental.pallas import tpu_sc as plsc

B, L, D = 16384, 200, 128
BT = B * L
NC, NS = 2, 16
NW = NC * NS
BPW = BT // NW
GSUB = 128
NQ = 2
CH = NQ * GSUB
NCH = BPW // CH
NROWS = 128


def _body(x_hbm, tab_hbm, out_hbm,
          tab_s, idx0, idx1, rows0, rows1,
          isem0, isem1, gsem0, gsem1, osem0, osem1):
    wid = lax.axis_index("s") * NC + lax.axis_index("c")
    base = wid * BPW

    @pl.when(lax.axis_index("s") == 0)
    def _():
        pltpu.sync_copy(tab_hbm.at[pl.ds(0, NROWS)], tab_s)

    plsc.subcore_barrier()
    idx = (idx0, idx1)
    rows = (rows0, rows1)
    isem = (isem0, isem1)
    gsem = (gsem0, gsem1)
    osem = (osem0, osem1)

    def fetch_idx(c, s):
        pltpu.async_copy(x_hbm.at[wid, c], idx[s], isem[s])

    def wait_idx(c, s):
        pltpu.make_async_copy(x_hbm.at[wid, c], idx[s], isem[s]).wait()

    def start_gathers(s):
        for q in range(NQ):
            pltpu.async_copy(tab_s.at[idx[s].at[q]],
                             rows[s].at[pl.ds(q * GSUB, GSUB)], gsem[s])

    def wait_gathers(s):
        for q in range(NQ):
            pltpu.make_async_copy(tab_s.at[idx[s].at[q]],
                                  rows[s].at[pl.ds(q * GSUB, GSUB)],
                                  gsem[s]).wait()

    def start_scatter(c, s):
        pltpu.async_copy(rows[s], out_hbm.at[pl.ds(base + c * CH, CH)],
                         osem[s])

    def wait_scatter(c, s):
        pltpu.make_async_copy(rows[s], out_hbm.at[pl.ds(base + c * CH, CH)],
                              osem[s]).wait()

    fetch_idx(0, 0)
    fetch_idx(1, 1)
    wait_idx(0, 0)
    start_gathers(0)

    def step(g, _):
        c0 = 2 * g
        wait_gathers(0)
        start_scatter(c0, 0)

        @pl.when(g <= NCH // 2 - 2)
        def _():
            fetch_idx(c0 + 2, 0)

        @pl.when(g >= 1)
        def _():
            wait_scatter(c0 - 1, 1)

        wait_idx(c0 + 1, 1)
        start_gathers(1)

        wait_gathers(1)
        start_scatter(c0 + 1, 1)

        @pl.when(g <= NCH // 2 - 2)
        def _():
            fetch_idx(c0 + 3, 1)
            wait_scatter(c0, 0)
            wait_idx(c0 + 2, 0)
            start_gathers(0)

        return 0

    lax.fori_loop(0, NCH // 2, step, 0)

    wait_scatter(NCH - 2, 0)
    wait_scatter(NCH - 1, 1)


_mesh = plsc.VectorSubcoreMesh(core_axis_name="c", subcore_axis_name="s")

_sc_gather = pl.kernel(
    _body,
    out_type=jax.ShapeDtypeStruct((BT, D), jnp.float32),
    mesh=_mesh,
    scratch_types=[
        pltpu.VMEM_SHARED((NROWS, D), jnp.float32),
        pltpu.VMEM((NQ, GSUB), jnp.int32),
        pltpu.VMEM((NQ, GSUB), jnp.int32),
        pltpu.VMEM((CH, D), jnp.float32),
        pltpu.VMEM((CH, D), jnp.float32),
        pltpu.SemaphoreType.DMA,
        pltpu.SemaphoreType.DMA,
        pltpu.SemaphoreType.DMA,
        pltpu.SemaphoreType.DMA,
        pltpu.SemaphoreType.DMA,
        pltpu.SemaphoreType.DMA,
    ],
)


@jax.jit
def kernel(x, pe):
    xr = x.reshape(NW, NCH, NQ, GSUB)
    out = _sc_gather(xr, pe)
    return out.reshape(B, L, D)
